# baseline (speedup 1.0000x reference)
import functools

import jax
import jax.numpy as jnp
from jax import lax
from jax.experimental import pallas as pl
from jax.experimental.pallas import tpu as pltpu
from jax.experimental.pallas import tpu_sc as plsc

N = 50000
NOUT = 51200
E = 800000
HIDDEN = 64

CHUNK = 128
NW = 32
EPAD = 819200
ACC_ROWS = NOUT
EPI = NOUT // 16
IDX_PAD = 16

_f32 = jnp.float32


def _make_agg(width, split_edges, sup, do_gather=True):
  mesh = plsc.VectorSubcoreMesh(
      core_axis_name="c", subcore_axis_name="s", num_cores=2, num_subcores=16)
  per_tile = EPAD // NW if split_edges else EPAD // 16
  blocks = per_tile // (sup * CHUNK)
  pairs = blocks // 2
  assert blocks % 2 == 0

  @functools.partial(
      pl.kernel,
      out_type=[jax.ShapeDtypeStruct((NOUT, width), _f32)] * 2,
      mesh=mesh,
      compiler_params=pltpu.CompilerParams(use_tc_tiling_on_sc=False),
      scratch_types=(
          [pltpu.VMEM((CHUNK,), jnp.int32) for _ in range(4 * sup)] + [
              pltpu.VMEM((sup * CHUNK, width), _f32),
              pltpu.VMEM((sup * CHUNK, width), _f32),
              pltpu.VMEM_SHARED((ACC_ROWS, width), _f32),
          ] + [pltpu.SemaphoreType.DMA] * 6),
  )
  def agg(src_hbm, dst_hbm, tbl_a, tbl_b, zeros_hbm, out_a, out_b, *scr):
    src_i = (scr[:sup], scr[sup:2 * sup])
    dst_i = (scr[2 * sup:3 * sup], scr[3 * sup:4 * sup])
    rows = (scr[4 * sup], scr[4 * sup + 1])
    acc = scr[4 * sup + 2]
    isem = scr[4 * sup + 3:4 * sup + 5]
    gsem = scr[4 * sup + 5:4 * sup + 7]
    ssem = scr[4 * sup + 7:4 * sup + 9]
    c = lax.axis_index("c")
    s = lax.axis_index("s")

    pltpu.sync_copy(zeros_hbm, acc.at[pl.ds(s * EPI, EPI)])
    plsc.subcore_barrier()

    def load_idx(blk, p):
      for j in range(sup):
        pltpu.async_copy(dst_hbm.at[blk * sup + j], dst_i[p][j], isem[p])
        if do_gather:
          pltpu.async_copy(src_hbm.at[blk * sup + j], src_i[p][j], isem[p])

    def drain_idx(p):
      for j in range(sup):
        pltpu.make_async_copy(dst_hbm.at[j], dst_i[p][j], isem[p]).wait()
        if do_gather:
          pltpu.make_async_copy(src_hbm.at[j], src_i[p][j], isem[p]).wait()

    def run(tbl, out):
      if split_edges:
        base = (s * 2 + c) * blocks
      else:
        base = s * blocks

      def fire_gather(p):
        for j in range(sup):
          pltpu.async_copy(
              tbl.at[src_i[p][j]],
              rows[p].at[pl.ds(j * CHUNK, CHUNK)], gsem[p])

      def drain_gather(p):
        for j in range(sup):
          pltpu.make_async_copy(
              tbl.at[pl.ds(0, CHUNK)],
              rows[p].at[pl.ds(j * CHUNK, CHUNK)], gsem[p]).wait()

      def fire_scatter(p, rp):
        return [
            pltpu.async_copy(
                rows[rp].at[pl.ds(j * CHUNK, CHUNK)],
                acc.at[dst_i[p][j]], ssem[p], add=True)
            for j in range(sup)
        ]

      if do_gather:
        load_idx(base, 0)
        load_idx(base + 1, 1)
        drain_idx(0)
        fire_gather(0)

        def body(q, carry):
          e = base + 2 * q
          drain_gather(0)
          drain_idx(1)
          fire_gather(1)
          for d in fire_scatter(0, 0):
            d.wait()
          load_idx(e + 2, 0)
          drain_gather(1)
          drain_idx(0)
          fire_gather(0)
          for d in fire_scatter(1, 1):
            d.wait()
          load_idx(e + 3, 1)
          return carry

        lax.fori_loop(0, pairs, body, 0)
        drain_gather(0)
        drain_idx(1)
      else:
        pltpu.sync_copy(tbl, rows[0])
        load_idx(base, 0)
        load_idx(base + 1, 1)

        def body(q, carry):
          e = base + 2 * q
          for p in range(2):
            drain_idx(p)
            puts = fire_scatter(p, 0)
            for d in puts:
              d.wait()
            load_idx(e + 2 + p, p)
          return carry

        lax.fori_loop(0, pairs, body, 0)
        drain_idx(0)
        drain_idx(1)
      plsc.subcore_barrier()

      pltpu.sync_copy(acc.at[pl.ds(s * EPI, EPI)], out.at[pl.ds(s * EPI, EPI)])

    @pl.when(c == 0)
    def _():
      run(tbl_a, out_a)

    @pl.when(c == 1)
    def _():
      run(tbl_b, out_b)

  return agg


_deg1 = _make_agg(8, split_edges=True, sup=4, do_gather=False)
_agg2 = _make_agg(8, split_edges=True, sup=4)
_agg32 = _make_agg(32, split_edges=False, sup=2)


def _scale_kernel(dega, degb, xpad, dinv_o, g1_o):
  deg = dega[...][:, :1] + degb[...][:, :1] + 1.0
  dinv = lax.rsqrt(deg)
  dinv_o[...] = dinv
  g1_o[...] = xpad[...] * dinv


def _layer1_kernel(s1a, s1b, g1, dinv, w1, b1, g2a_o, g2b_o):
  z = dinv[...] * (s1a[...] + s1b[...] + g1[...])
  h = (z[:, 0:1] * w1[0:1, :] + z[:, 1:2] * w1[1:2, :]) + b1[...]
  g2 = dinv[...] * jnp.maximum(h, 0.0)
  g2a_o[...] = g2[:, :32]
  g2b_o[...] = g2[:, 32:]


def _layer2_kernel(s2a, s2b, g2a, g2b, dinv, w2, b2, out_o):
  za = dinv[...] * (s2a[...] + g2a[...])
  zb = dinv[...] * (s2b[...] + g2b[...])
  h = (jnp.dot(za, w2[:32, :], preferred_element_type=_f32)
       + jnp.dot(zb, w2[32:, :], preferred_element_type=_f32)) + b2[...]
  out_o[...] = jnp.maximum(h, 0.0)


_R = 2048


def kernel(x, edge_index, W1, b1, W2, b2):
  src = edge_index[0].astype(jnp.int32)
  dst = edge_index[1].astype(jnp.int32)
  pad = EPAD - E
  extra = IDX_PAD * CHUNK
  src2d = jnp.concatenate(
      [src, jnp.zeros((pad + extra,), jnp.int32)]).reshape(-1, CHUNK)
  dst2d = jnp.concatenate(
      [dst, jnp.full((pad,), N, jnp.int32),
       jnp.zeros((extra,), jnp.int32)]).reshape(-1, CHUNK)

  zeros8 = jnp.zeros((EPI, 8), _f32)
  zeros32 = jnp.zeros((EPI, 32), _f32)
  ones_blk = jnp.ones((4 * CHUNK, 8), _f32)
  xpad = jnp.zeros((NOUT, 8), _f32).at[:N, :2].set(x)

  dega, degb = _deg1(src2d, dst2d, ones_blk, ones_blk, zeros8)

  nb = NOUT // _R
  dinv, g1 = pl.pallas_call(
      _scale_kernel,
      grid=(nb,),
      in_specs=[
          pl.BlockSpec((_R, 8), lambda i: (i, 0)),
          pl.BlockSpec((_R, 8), lambda i: (i, 0)),
          pl.BlockSpec((_R, 8), lambda i: (i, 0)),
      ],
      out_specs=[
          pl.BlockSpec((_R, 1), lambda i: (i, 0)),
          pl.BlockSpec((_R, 8), lambda i: (i, 0)),
      ],
      out_shape=[
          jax.ShapeDtypeStruct((NOUT, 1), _f32),
          jax.ShapeDtypeStruct((NOUT, 8), _f32),
      ],
  )(dega, degb, xpad)

  s1a, s1b = _agg2(src2d, dst2d, g1, g1, zeros8)
  g2a, g2b = pl.pallas_call(
      _layer1_kernel,
      grid=(nb,),
      in_specs=[
          pl.BlockSpec((_R, 8), lambda i: (i, 0)),
          pl.BlockSpec((_R, 8), lambda i: (i, 0)),
          pl.BlockSpec((_R, 8), lambda i: (i, 0)),
          pl.BlockSpec((_R, 1), lambda i: (i, 0)),
          pl.BlockSpec((2, HIDDEN), lambda i: (0, 0)),
          pl.BlockSpec((1, HIDDEN), lambda i: (0, 0)),
      ],
      out_specs=[
          pl.BlockSpec((_R, 32), lambda i: (i, 0)),
          pl.BlockSpec((_R, 32), lambda i: (i, 0)),
      ],
      out_shape=[jax.ShapeDtypeStruct((NOUT, 32), _f32)] * 2,
  )(s1a, s1b, g1, dinv, W1, b1.reshape(1, HIDDEN))

  s2a, s2b = _agg32(src2d, dst2d, g2a, g2b, zeros32)

  out = pl.pallas_call(
      _layer2_kernel,
      grid=(nb,),
      in_specs=[
          pl.BlockSpec((_R, 32), lambda i: (i, 0)),
          pl.BlockSpec((_R, 32), lambda i: (i, 0)),
          pl.BlockSpec((_R, 32), lambda i: (i, 0)),
          pl.BlockSpec((_R, 32), lambda i: (i, 0)),
          pl.BlockSpec((_R, 1), lambda i: (i, 0)),
          pl.BlockSpec((HIDDEN, HIDDEN), lambda i: (0, 0)),
          pl.BlockSpec((1, HIDDEN), lambda i: (0, 0)),
      ],
      out_specs=pl.BlockSpec((_R, HIDDEN), lambda i: (i, 0)),
      out_shape=jax.ShapeDtypeStruct((NOUT, HIDDEN), _f32),
  )(s2a, s2b, g2a, g2b, dinv, W2, b2.reshape(1, HIDDEN))

  return out[:N]


# speedup vs baseline: 1.0048x; 1.0048x over previous
import functools

import jax
import jax.numpy as jnp
from jax import lax
from jax.experimental import pallas as pl
from jax.experimental.pallas import tpu as pltpu
from jax.experimental.pallas import tpu_sc as plsc

N = 50000
NOUT = 51200
E = 800000
HIDDEN = 64

CHUNK = 128
NW = 32
EPAD = 819200
ACC_ROWS = NOUT
EPI = NOUT // 16
IDX_PAD = 16

_f32 = jnp.float32


def _make_agg(width, split_edges, sup, do_gather=True):
  mesh = plsc.VectorSubcoreMesh(
      core_axis_name="c", subcore_axis_name="s", num_cores=2, num_subcores=16)
  per_tile = EPAD // NW if split_edges else EPAD // 16
  blocks = per_tile // (sup * CHUNK)
  pairs = blocks // 2
  assert blocks % 2 == 0

  @functools.partial(
      pl.kernel,
      out_type=[jax.ShapeDtypeStruct((NOUT, width), _f32)] * 2,
      mesh=mesh,
      compiler_params=pltpu.CompilerParams(use_tc_tiling_on_sc=False),
      scratch_types=(
          [pltpu.VMEM((CHUNK,), jnp.int32) for _ in range(4 * sup)] + [
              pltpu.VMEM((sup * CHUNK, width), _f32),
              pltpu.VMEM((sup * CHUNK, width), _f32),
              pltpu.VMEM_SHARED((ACC_ROWS, width), _f32),
          ] + [pltpu.SemaphoreType.DMA] * 6),
  )
  def agg(src_hbm, dst_hbm, tbl_a, tbl_b, zeros_hbm, out_a, out_b, *scr):
    src_i = (scr[:sup], scr[sup:2 * sup])
    dst_i = (scr[2 * sup:3 * sup], scr[3 * sup:4 * sup])
    rows = (scr[4 * sup], scr[4 * sup + 1])
    acc = scr[4 * sup + 2]
    isem = scr[4 * sup + 3:4 * sup + 5]
    gsem = scr[4 * sup + 5:4 * sup + 7]
    ssem = scr[4 * sup + 7:4 * sup + 9]
    c = lax.axis_index("c")
    s = lax.axis_index("s")

    pltpu.sync_copy(zeros_hbm, acc.at[pl.ds(s * EPI, EPI)])
    plsc.subcore_barrier()

    def load_idx(blk, p):
      for j in range(sup):
        pltpu.async_copy(dst_hbm.at[blk * sup + j], dst_i[p][j], isem[p])
        if do_gather:
          pltpu.async_copy(src_hbm.at[blk * sup + j], src_i[p][j], isem[p])

    def drain_idx(p):
      for j in range(sup):
        pltpu.make_async_copy(dst_hbm.at[j], dst_i[p][j], isem[p]).wait()
        if do_gather:
          pltpu.make_async_copy(src_hbm.at[j], src_i[p][j], isem[p]).wait()

    def run(tbl, out):
      if split_edges:
        base = (s * 2 + c) * blocks
      else:
        base = s * blocks

      def fire_gather(p):
        for j in range(sup):
          pltpu.async_copy(
              tbl.at[src_i[p][j]],
              rows[p].at[pl.ds(j * CHUNK, CHUNK)], gsem[p])

      def drain_gather(p):
        for j in range(sup):
          pltpu.make_async_copy(
              tbl.at[pl.ds(0, CHUNK)],
              rows[p].at[pl.ds(j * CHUNK, CHUNK)], gsem[p]).wait()

      def fire_scatter(p, rp):
        return [
            pltpu.async_copy(
                rows[rp].at[pl.ds(j * CHUNK, CHUNK)],
                acc.at[dst_i[p][j]], ssem[p], add=True)
            for j in range(sup)
        ]

      if do_gather:
        load_idx(base, 0)
        load_idx(base + 1, 1)
        drain_idx(0)
        fire_gather(0)

        def body(q, carry):
          e = base + 2 * q
          drain_gather(0)
          drain_idx(1)
          fire_gather(1)
          for d in fire_scatter(0, 0):
            d.wait()
          load_idx(e + 2, 0)
          drain_gather(1)
          drain_idx(0)
          fire_gather(0)
          for d in fire_scatter(1, 1):
            d.wait()
          load_idx(e + 3, 1)
          return carry

        lax.fori_loop(0, pairs, body, 0)
        drain_gather(0)
        drain_idx(1)
      else:
        pltpu.sync_copy(tbl, rows[0])
        load_idx(base, 0)
        load_idx(base + 1, 1)

        def body(q, carry):
          e = base + 2 * q
          for p in range(2):
            drain_idx(p)
            puts = fire_scatter(p, 0)
            for d in puts:
              d.wait()
            load_idx(e + 2 + p, p)
          return carry

        lax.fori_loop(0, pairs, body, 0)
        drain_idx(0)
        drain_idx(1)
      plsc.subcore_barrier()

      pltpu.sync_copy(acc.at[pl.ds(s * EPI, EPI)], out.at[pl.ds(s * EPI, EPI)])

    @pl.when(c == 0)
    def _():
      run(tbl_a, out_a)

    @pl.when(c == 1)
    def _():
      run(tbl_b, out_b)

  return agg


_deg1 = _make_agg(8, split_edges=True, sup=4, do_gather=False)
_agg2 = _make_agg(8, split_edges=True, sup=4)
_agg32 = _make_agg(32, split_edges=False, sup=2)


def _scale_kernel(dega, degb, xpad, dinv_o, g1_o):
  deg = dega[...][:, :1] + degb[...][:, :1] + 1.0
  dinv = lax.rsqrt(deg)
  dinv_o[...] = dinv
  g1_o[...] = xpad[...] * dinv


def _layer1_kernel(s1a, s1b, g1, dinv, w1, b1, g2a_o, g2b_o):
  z = dinv[...] * (s1a[...] + s1b[...] + g1[...])
  h = (z[:, 0:1] * w1[0:1, :] + z[:, 1:2] * w1[1:2, :]) + b1[...]
  g2 = dinv[...] * jnp.maximum(h, 0.0)
  g2a_o[...] = g2[:, :32]
  g2b_o[...] = g2[:, 32:]


def _layer2_kernel(s2a, s2b, g2a, g2b, dinv, w2, b2, out_o):
  za = dinv[...] * (s2a[...] + g2a[...])
  zb = dinv[...] * (s2b[...] + g2b[...])
  h = (jnp.dot(za, w2[:32, :], preferred_element_type=_f32)
       + jnp.dot(zb, w2[32:, :], preferred_element_type=_f32)) + b2[...]
  out_o[...] = jnp.maximum(h, 0.0)


_R = 6400


def kernel(x, edge_index, W1, b1, W2, b2):
  src = edge_index[0].astype(jnp.int32)
  dst = edge_index[1].astype(jnp.int32)
  pad = EPAD - E
  extra = IDX_PAD * CHUNK
  src2d = jnp.concatenate(
      [src, jnp.zeros((pad + extra,), jnp.int32)]).reshape(-1, CHUNK)
  dst2d = jnp.concatenate(
      [dst, jnp.full((pad,), N, jnp.int32),
       jnp.zeros((extra,), jnp.int32)]).reshape(-1, CHUNK)

  zeros8 = jnp.zeros((EPI, 8), _f32)
  zeros32 = jnp.zeros((EPI, 32), _f32)
  ones_blk = jnp.ones((4 * CHUNK, 8), _f32)
  xpad = jnp.zeros((NOUT, 8), _f32).at[:N, :2].set(x)

  dega, degb = _deg1(src2d, dst2d, ones_blk, ones_blk, zeros8)

  nb = NOUT // _R
  dinv, g1 = pl.pallas_call(
      _scale_kernel,
      grid=(nb,),
      in_specs=[
          pl.BlockSpec((_R, 8), lambda i: (i, 0)),
          pl.BlockSpec((_R, 8), lambda i: (i, 0)),
          pl.BlockSpec((_R, 8), lambda i: (i, 0)),
      ],
      out_specs=[
          pl.BlockSpec((_R, 1), lambda i: (i, 0)),
          pl.BlockSpec((_R, 8), lambda i: (i, 0)),
      ],
      out_shape=[
          jax.ShapeDtypeStruct((NOUT, 1), _f32),
          jax.ShapeDtypeStruct((NOUT, 8), _f32),
      ],
  )(dega, degb, xpad)

  s1a, s1b = _agg2(src2d, dst2d, g1, g1, zeros8)
  g2a, g2b = pl.pallas_call(
      _layer1_kernel,
      grid=(nb,),
      in_specs=[
          pl.BlockSpec((_R, 8), lambda i: (i, 0)),
          pl.BlockSpec((_R, 8), lambda i: (i, 0)),
          pl.BlockSpec((_R, 8), lambda i: (i, 0)),
          pl.BlockSpec((_R, 1), lambda i: (i, 0)),
          pl.BlockSpec((2, HIDDEN), lambda i: (0, 0)),
          pl.BlockSpec((1, HIDDEN), lambda i: (0, 0)),
      ],
      out_specs=[
          pl.BlockSpec((_R, 32), lambda i: (i, 0)),
          pl.BlockSpec((_R, 32), lambda i: (i, 0)),
      ],
      out_shape=[jax.ShapeDtypeStruct((NOUT, 32), _f32)] * 2,
  )(s1a, s1b, g1, dinv, W1, b1.reshape(1, HIDDEN))

  s2a, s2b = _agg32(src2d, dst2d, g2a, g2b, zeros32)

  out = pl.pallas_call(
      _layer2_kernel,
      grid=(nb,),
      in_specs=[
          pl.BlockSpec((_R, 32), lambda i: (i, 0)),
          pl.BlockSpec((_R, 32), lambda i: (i, 0)),
          pl.BlockSpec((_R, 32), lambda i: (i, 0)),
          pl.BlockSpec((_R, 32), lambda i: (i, 0)),
          pl.BlockSpec((_R, 1), lambda i: (i, 0)),
          pl.BlockSpec((HIDDEN, HIDDEN), lambda i: (0, 0)),
          pl.BlockSpec((1, HIDDEN), lambda i: (0, 0)),
      ],
      out_specs=pl.BlockSpec((_R, HIDDEN), lambda i: (i, 0)),
      out_shape=jax.ShapeDtypeStruct((NOUT, HIDDEN), _f32),
  )(s2a, s2b, g2a, g2b, dinv, W2, b2.reshape(1, HIDDEN))

  return out[:N]
